# Initial kernel scaffold; baseline (speedup 1.0000x reference)
import jax
import jax.numpy as jnp
from jax.experimental import pallas as pl


def kernel(logits, targets):
    raise NotImplementedError("write your pallas kernel here")


# speedup vs baseline: 3.8713x; 3.8713x over previous
import functools

import jax
import jax.numpy as jnp
from jax import lax
from jax.experimental import pallas as pl

N = 8192
V = 8192
K = 2048
TN = 256
GRID = N // TN

_INTERPRET = False


def _stats_body(targets_ref, logits_ref, keys_ref, nll_ref):
    x = logits_ref[...]
    t = targets_ref[...]
    col = lax.broadcasted_iota(jnp.int32, (TN, V), 1)
    tmask = col == t
    m = jnp.max(x, axis=1, keepdims=True)
    xt = jnp.sum(jnp.where(tmask, x, 0.0), axis=1, keepdims=True)
    xm = x - m
    e = jnp.exp(xm)
    s = jnp.sum(e, axis=1, keepdims=True)
    w = jnp.sum(e * xm, axis=1, keepdims=True)
    logs = jnp.log(s)
    ent = logs - w / s
    nll = (m + logs) - xt
    b = lax.bitcast_convert_type(ent, jnp.int32)
    keys_ref[...] = jnp.where(b < 0, b ^ 0x7FFFFFFF, b)
    nll_ref[...] = nll


def _stage1(logits_flat, targets_col):
    return pl.pallas_call(
        _stats_body,
        grid=(GRID,),
        in_specs=[
            pl.BlockSpec((TN, 1), lambda i: (i, 0)),
            pl.BlockSpec((TN, V), lambda i: (i, 0)),
        ],
        out_specs=[
            pl.BlockSpec((TN, 1), lambda i: (i, 0)),
            pl.BlockSpec((TN, 1), lambda i: (i, 0)),
        ],
        out_shape=[
            jax.ShapeDtypeStruct((N, 1), jnp.int32),
            jax.ShapeDtypeStruct((N, 1), jnp.float32),
        ],
        interpret=_INTERPRET,
    )(targets_col, logits_flat)


def kernel(logits, targets):
    logits_flat = logits.reshape(N, V)
    targets_col = targets.reshape(N, 1)
    keys, nll = _stage1(logits_flat, targets_col)
    keys = keys.reshape(N)
    nll = nll.reshape(N)
    _, sel = lax.top_k(keys, K)
    return jnp.mean(jnp.take(nll, sel))
